# baseline (speedup 1.0000x reference)
import jax
import jax.numpy as jnp
from jax.experimental import pallas as pl

_N = 10000
_SLACK_BUS = 0
_SLACK_VOLTAGE = 1.0
_SLACK_ANGLE = 0.0
_V_MIN, _V_MAX = 0.95, 1.05
_POWER_BALANCE_WEIGHT = 10.0


def _edge_math(r_ref, x_ref, vi_ref, vj_ref, th_ref, m_ref, p_ref, q_ref):
    r = r_ref[...]
    x = x_ref[...]
    z2 = r * r + x * x
    g = r / z2
    b = -x / z2
    c = jnp.cos(th_ref[...])
    s = jnp.sin(th_ref[...])
    vij = vi_ref[...] * vj_ref[...] * m_ref[...]
    p_ref[...] = vij * (g * c + b * s)
    q_ref[...] = vij * (g * s - b * c)


def kernel(v_mag, v_ang, r_line, x_line, edge_index):
    ei = edge_index.astype(jnp.int32)
    from_bus = ei[0]
    to_bus = ei[1]
    a = jnp.minimum(from_bus, to_bus)
    b = jnp.maximum(from_bus, to_bus)
    keys = a * _N + b
    order = jnp.argsort(keys, stable=True)
    sorted_keys = keys[order]
    first_sorted = jnp.concatenate(
        [jnp.ones((1,), dtype=bool), sorted_keys[1:] != sorted_keys[:-1]]
    )
    is_first = jnp.zeros((keys.shape[0],), dtype=bool).at[order].set(first_sorted)
    rank = jnp.cumsum(is_first.astype(jnp.int32)) - 1
    col = jnp.clip(rank, 0, None)

    r = r_line[:, col]
    x = x_line[:, col]
    v_i = v_mag[:, from_bus]
    v_j = v_mag[:, to_bus]
    theta = v_ang[:, from_bus] - v_ang[:, to_bus]
    mask = jnp.broadcast_to(is_first.astype(v_mag.dtype), r.shape)

    B, E = r.shape
    p_ij, q_ij = pl.pallas_call(
        _edge_math,
        out_shape=(
            jax.ShapeDtypeStruct((B, E), jnp.float32),
            jax.ShapeDtypeStruct((B, E), jnp.float32),
        ),
    )(r, x, v_i, v_j, theta, mask)

    p_calc = jnp.zeros_like(v_mag).at[:, from_bus].add(p_ij)
    q_calc = jnp.zeros_like(v_mag).at[:, from_bus].add(q_ij)
    p_mismatch = p_calc.at[:, _SLACK_BUS].set(0.0)
    q_mismatch = q_calc.at[:, _SLACK_BUS].set(0.0)
    power_loss = jnp.mean(p_mismatch**2 + q_mismatch**2)
    lower = jax.nn.relu(_V_MIN - v_mag)
    upper = jax.nn.relu(v_mag - _V_MAX)
    voltage_penalty = jnp.mean(jnp.sum(lower**2 + upper**2, axis=-1))
    slack_v = v_mag[:, _SLACK_BUS]
    slack_a = v_ang[:, _SLACK_BUS]
    slack_penalty = jnp.mean((slack_v - _SLACK_VOLTAGE) ** 2 + (slack_a - _SLACK_ANGLE) ** 2)
    constraint_loss = _POWER_BALANCE_WEIGHT * power_loss + voltage_penalty + slack_penalty
    return (v_mag, v_ang, constraint_loss)
